# baseline (device time: 10083 ns/iter reference)
import jax
import jax.numpy as jnp
from jax import lax
from jax.experimental import pallas as pl
from jax.experimental.pallas import tpu as pltpu

M = 512
D = 512
NCHUNK = 4
CM = M // NCHUNK


def kernel(partial, gamma):
    gam = gamma.reshape(1, D)

    def body(
        part_ref,
        gamma_ref,
        out_ref,
        send_stage,
        mine_stage,
        send_q,
        recv_q,
        out_stage,
        send_in_sems,
        mine_in_sems,
        out_sems,
        send_sems,
        recv_sems,
    ):
        my_x = lax.axis_index("x")
        my_y = lax.axis_index("y")
        my_z = lax.axis_index("z")
        other = 1 - my_x

        barrier = pltpu.get_barrier_semaphore()
        pl.semaphore_signal(
            barrier,
            inc=1,
            device_id=(other, my_y, my_z),
            device_id_type=pl.DeviceIdType.MESH,
        )

        send_in = []
        mine_in = []
        for c in range(NCHUNK):
            cp = pltpu.make_async_copy(
                part_ref.at[0, pl.ds(other * M + c * CM, CM), :],
                send_stage.at[c],
                send_in_sems.at[c],
            )
            cp.start()
            send_in.append(cp)
        for c in range(NCHUNK):
            cp = pltpu.make_async_copy(
                part_ref.at[0, pl.ds(my_x * M + c * CM, CM), :],
                mine_stage.at[c],
                mine_in_sems.at[c],
            )
            cp.start()
            mine_in.append(cp)

        send_in[0].wait()
        send_q[0] = send_stage[0].astype(jnp.float8_e4m3fn)
        pl.semaphore_wait(barrier, 1)

        rdmas = []
        for c in range(NCHUNK):
            rdma = pltpu.make_async_remote_copy(
                src_ref=send_q.at[c],
                dst_ref=recv_q.at[c],
                send_sem=send_sems.at[c],
                recv_sem=recv_sems.at[c],
                device_id=(other, my_y, my_z),
                device_id_type=pl.DeviceIdType.MESH,
            )
            rdma.start()
            rdmas.append(rdma)
            if c + 1 < NCHUNK:
                send_in[c + 1].wait()
                send_q[c + 1] = send_stage[c + 1].astype(jnp.float8_e4m3fn)

        out_cps = []
        for c in range(NCHUNK):
            rdmas[c].wait_recv()
            mine_in[c].wait()
            y = mine_stage[c] + recv_q[c].astype(jnp.float32)
            rms = jnp.sqrt(jnp.mean(y * y, axis=-1, keepdims=True) + 1e-6)
            out_stage[c] = y / rms * gamma_ref[:, :]
            cp = pltpu.make_async_copy(
                out_stage.at[c],
                out_ref.at[pl.ds(c * CM, CM), :],
                out_sems.at[c],
            )
            cp.start()
            out_cps.append(cp)

        for c in range(NCHUNK):
            out_cps[c].wait()
            rdmas[c].wait_send()

    return pl.pallas_call(
        body,
        out_shape=jax.ShapeDtypeStruct((M, D), jnp.float32),
        in_specs=[
            pl.BlockSpec(memory_space=pl.ANY),
            pl.BlockSpec(memory_space=pltpu.VMEM),
        ],
        out_specs=pl.BlockSpec(memory_space=pl.ANY),
        scratch_shapes=[
            pltpu.VMEM((NCHUNK, CM, D), jnp.float32),
            pltpu.VMEM((NCHUNK, CM, D), jnp.float32),
            pltpu.VMEM((NCHUNK, CM, D), jnp.float8_e4m3fn),
            pltpu.VMEM((NCHUNK, CM, D), jnp.float8_e4m3fn),
            pltpu.VMEM((NCHUNK, CM, D), jnp.float32),
            pltpu.SemaphoreType.DMA((NCHUNK,)),
            pltpu.SemaphoreType.DMA((NCHUNK,)),
            pltpu.SemaphoreType.DMA((NCHUNK,)),
            pltpu.SemaphoreType.DMA((NCHUNK,)),
            pltpu.SemaphoreType.DMA((NCHUNK,)),
        ],
        compiler_params=pltpu.CompilerParams(collective_id=0),
    )(partial, gam)


# device time: 9769 ns/iter; 1.0321x vs baseline; 1.0321x over previous
import jax
import jax.numpy as jnp
from jax import lax
from jax.experimental import pallas as pl
from jax.experimental.pallas import tpu as pltpu

M = 512
D = 512
NCHUNK = 4
CM = M // NCHUNK
NFP8 = 4


def _wire_dtype(c):
    return jnp.float8_e4m3fn if c < NFP8 else jnp.bfloat16


def kernel(partial, gamma):
    gam = gamma.reshape(1, D)

    def body(
        part_ref,
        gamma_ref,
        out_ref,
        send_q,
        recv_q,
        send_h,
        recv_h,
        send_sems,
        recv_sems,
    ):
        my_x = lax.axis_index("x")
        my_y = lax.axis_index("y")
        my_z = lax.axis_index("z")
        other = 1 - my_x

        def bufs(c):
            return (send_q.at[c], recv_q.at[c]) if c < NFP8 else (
                send_h.at[c - NFP8],
                recv_h.at[c - NFP8],
            )

        def stage(c):
            rows = part_ref[0, pl.ds(other * M + c * CM, CM), :]
            if c < NFP8:
                send_q[c] = rows.astype(jnp.float8_e4m3fn)
            else:
                send_h[c - NFP8] = rows.astype(jnp.bfloat16)

        barrier = pltpu.get_barrier_semaphore()
        pl.semaphore_signal(
            barrier,
            inc=1,
            device_id=(other, my_y, my_z),
            device_id_type=pl.DeviceIdType.MESH,
        )
        stage(0)
        pl.semaphore_wait(barrier, 1)

        rdmas = []
        for c in range(NCHUNK):
            src, dst = bufs(c)
            rdma = pltpu.make_async_remote_copy(
                src_ref=src,
                dst_ref=dst,
                send_sem=send_sems.at[c],
                recv_sem=recv_sems.at[c],
                device_id=(other, my_y, my_z),
                device_id_type=pl.DeviceIdType.MESH,
            )
            rdma.start()
            rdmas.append(rdma)
            if c + 1 < NCHUNK:
                stage(c + 1)

        for c in range(NCHUNK):
            rdmas[c].wait_recv()
            recv = recv_q[c] if c < NFP8 else recv_h[c - NFP8]
            y = part_ref[0, pl.ds(my_x * M + c * CM, CM), :] + recv.astype(
                jnp.float32
            )
            rms = jnp.sqrt(jnp.mean(y * y, axis=-1, keepdims=True) + 1e-6)
            out_ref[pl.ds(c * CM, CM), :] = y / rms * gamma_ref[:, :]

        for c in range(NCHUNK):
            rdmas[c].wait_send()

    nh = max(NCHUNK - NFP8, 1)
    return pl.pallas_call(
        body,
        out_shape=jax.ShapeDtypeStruct((M, D), jnp.float32),
        in_specs=[
            pl.BlockSpec(memory_space=pltpu.VMEM),
            pl.BlockSpec(memory_space=pltpu.VMEM),
        ],
        out_specs=pl.BlockSpec(memory_space=pltpu.VMEM),
        scratch_shapes=[
            pltpu.VMEM((max(NFP8, 1), CM, D), jnp.float8_e4m3fn),
            pltpu.VMEM((max(NFP8, 1), CM, D), jnp.float8_e4m3fn),
            pltpu.VMEM((nh, CM, D), jnp.bfloat16),
            pltpu.VMEM((nh, CM, D), jnp.bfloat16),
            pltpu.SemaphoreType.DMA((NCHUNK,)),
            pltpu.SemaphoreType.DMA((NCHUNK,)),
        ],
        compiler_params=pltpu.CompilerParams(collective_id=0),
    )(partial, gam)
